# baseline (speedup 1.0000x reference)
import functools

import jax
import jax.numpy as jnp
from jax import lax
from jax.experimental import pallas as pl
from jax.experimental.pallas import tpu as pltpu
from jax.experimental.pallas import tpu_sc as plsc

_NC, _NS = 2, 16
_NW = _NC * _NS
_BBLK = 128
_NBUF = 10
_LAG = 5
_CBUF = 4
_CLAG = 2

_mesh = plsc.VectorSubcoreMesh(core_axis_name="c", subcore_axis_name="s")


@functools.lru_cache(maxsize=None)
def _build_stage(batch: int, hist: int, embed_dim: int):
    assert batch == _NW * _BBLK and hist % _NBUF == 0
    bpw = batch // _NW
    assert bpw % _CBUF == 0

    @functools.partial(
        pl.kernel,
        mesh=_mesh,
        out_type=jax.ShapeDtypeStruct((_NW, hist, _BBLK), jnp.int32),
        scratch_types=[pltpu.VMEM((hist, _BBLK), jnp.int32)],
        compiler_params=pltpu.CompilerParams(use_tc_tiling_on_sc=True),
    )
    def stage_idx(xt_hbm, out_hbm, idx_v):
        wid = lax.axis_index("s") * _NC + lax.axis_index("c")
        pltpu.sync_copy(xt_hbm.at[:, pl.ds(wid * _BBLK, _BBLK)], idx_v)
        pltpu.sync_copy(idx_v, out_hbm.at[wid])

    gather_scratch = (
        [pltpu.VMEM((hist, _BBLK), jnp.int32)]
        + [pltpu.VMEM((_BBLK, embed_dim), jnp.float32)
           for _ in range(_NBUF)]
        + [pltpu.SemaphoreType.DMA for _ in range(2 * _NBUF)]
    )

    @functools.partial(
        pl.kernel,
        mesh=_mesh,
        out_type=jax.ShapeDtypeStruct((batch, hist, embed_dim), jnp.float32),
        scratch_types=gather_scratch,
        compiler_params=pltpu.CompilerParams(use_tc_tiling_on_sc=False),
    )
    def stage_gather(idx_hbm, table_hbm, out_hbm, idx_v, *bufs_and_sems):
        rows = bufs_and_sems[:_NBUF]
        gsem = bufs_and_sems[_NBUF:2 * _NBUF]
        osem = bufs_and_sems[2 * _NBUF:]

        wid = lax.axis_index("s") * _NC + lax.axis_index("c")
        b0 = wid * _BBLK

        pltpu.sync_copy(idx_hbm.at[wid], idx_v)

        def fire_gather(t, b):
            pltpu.make_async_copy(
                table_hbm.at[idx_v.at[t]], rows[b], gsem[b]).start()

        def wait_gather(b):
            pltpu.make_async_copy(
                table_hbm.at[idx_v.at[0]], rows[b], gsem[b]).wait()

        def fire_out(t, b):
            pltpu.make_async_copy(
                rows[b], out_hbm.at[pl.ds(b0, _BBLK), t], osem[b]).start()

        def wait_out(b):
            pltpu.make_async_copy(
                rows[b], out_hbm.at[pl.ds(b0, _BBLK), 0], osem[b]).wait()

        for b in range(_NBUF):
            fire_gather(b, b)
            if b >= _LAG:
                wait_gather(b - _LAG)
                fire_out(b - _LAG, b - _LAG)

        def body(g, _):
            t0 = g * _NBUF
            for b in range(_NBUF):
                t = t0 + b
                wait_out(b)
                fire_gather(t, b)
                b2 = (b - _LAG) % _NBUF
                wait_gather(b2)
                fire_out(t - _LAG, b2)
            return 0

        lax.fori_loop(1, hist // _NBUF, body, 0)

        tail0 = hist - _LAG
        for i in range(_LAG):
            b = (tail0 + i) % _NBUF
            wait_gather(b)
            fire_out(tail0 + i, b)
        for b in range(_NBUF):
            wait_out(b)

    relayout_scratch = (
        [pltpu.VMEM((hist, embed_dim), jnp.float32) for _ in range(_CBUF)]
        + [pltpu.SemaphoreType.DMA for _ in range(2 * _CBUF)]
    )

    @functools.partial(
        pl.kernel,
        mesh=_mesh,
        out_type=jax.ShapeDtypeStruct((batch, hist, embed_dim), jnp.float32),
        scratch_types=relayout_scratch,
        compiler_params=pltpu.CompilerParams(use_tc_tiling_on_sc=True),
    )
    def stage_relayout(in_hbm, out_hbm, *bufs_and_sems):
        bufs = bufs_and_sems[:_CBUF]
        rsem = bufs_and_sems[_CBUF:2 * _CBUF]
        wsem = bufs_and_sems[2 * _CBUF:]

        wid = lax.axis_index("s") * _NC + lax.axis_index("c")
        p0 = wid * bpw

        def fire_read(t, b):
            pltpu.make_async_copy(
                in_hbm.at[pl.ds((p0 + t) * hist, hist)], bufs[b],
                rsem[b]).start()

        def wait_read(b):
            pltpu.make_async_copy(
                in_hbm.at[pl.ds(0, hist)], bufs[b], rsem[b]).wait()

        def fire_write(t, b):
            pltpu.make_async_copy(bufs[b], out_hbm.at[p0 + t],
                                  wsem[b]).start()

        def wait_write(b):
            pltpu.make_async_copy(bufs[b], out_hbm.at[0], wsem[b]).wait()

        for b in range(_CBUF):
            fire_read(b, b)
            if b >= _CLAG:
                wait_read(b - _CLAG)
                fire_write(b - _CLAG, b - _CLAG)

        def body(g, _):
            t0 = g * _CBUF
            for b in range(_CBUF):
                t = t0 + b
                wait_write(b)
                fire_read(t, b)
                b2 = (b - _CLAG) % _CBUF
                wait_read(b2)
                fire_write(t - _CLAG, b2)
            return 0

        lax.fori_loop(1, bpw // _CBUF, body, 0)

        tail0 = bpw - _CLAG
        for i in range(_CLAG):
            b = (tail0 + i) % _CBUF
            wait_read(b)
            fire_write(tail0 + i, b)
        for b in range(_CBUF):
            wait_write(b)

    return stage_idx, stage_gather, stage_relayout


def kernel(x, table):
    b, h = x.shape
    d = table.shape[1]
    stage_idx, stage_gather, stage_relayout = _build_stage(b, h, d)
    xt = jnp.transpose(x)
    idx3 = stage_idx(xt)
    g = stage_gather(idx3, table)
    g2 = jnp.reshape(g, (b * h, d))
    return stage_relayout(g2)


# speedup vs baseline: 1.3245x; 1.3245x over previous
import functools

import jax
import jax.numpy as jnp
from jax import lax
from jax.experimental import pallas as pl
from jax.experimental.pallas import tpu as pltpu
from jax.experimental.pallas import tpu_sc as plsc

_NC, _NS = 2, 16
_NW = _NC * _NS
_BBLK = 128
_NBUF = 10
_LAG = 5

_mesh = plsc.VectorSubcoreMesh(core_axis_name="c", subcore_axis_name="s")


@functools.lru_cache(maxsize=None)
def _build_stage(batch: int, hist: int, embed_dim: int):
    assert batch == _NW * _BBLK and hist % _NBUF == 0

    @functools.partial(
        pl.kernel,
        mesh=_mesh,
        out_type=jax.ShapeDtypeStruct((_NW, hist, _BBLK), jnp.int32),
        scratch_types=[pltpu.VMEM((hist, _BBLK), jnp.int32)],
        compiler_params=pltpu.CompilerParams(use_tc_tiling_on_sc=True),
    )
    def stage_idx(xt_hbm, out_hbm, idx_v):
        wid = lax.axis_index("s") * _NC + lax.axis_index("c")
        pltpu.sync_copy(xt_hbm.at[:, pl.ds(wid * _BBLK, _BBLK)], idx_v)
        pltpu.sync_copy(idx_v, out_hbm.at[wid])

    gather_scratch = (
        [pltpu.VMEM((hist, _BBLK), jnp.int32)]
        + [pltpu.VMEM((_BBLK, embed_dim), jnp.float32)
           for _ in range(_NBUF)]
        + [pltpu.SemaphoreType.DMA for _ in range(2 * _NBUF)]
    )

    @functools.partial(
        pl.kernel,
        mesh=_mesh,
        out_type=jax.ShapeDtypeStruct((batch, hist, embed_dim), jnp.float32),
        scratch_types=gather_scratch,
        compiler_params=pltpu.CompilerParams(use_tc_tiling_on_sc=False),
    )
    def stage_gather(idx_hbm, table_hbm, out_hbm, idx_v, *bufs_and_sems):
        rows = bufs_and_sems[:_NBUF]
        gsem = bufs_and_sems[_NBUF:2 * _NBUF]
        osem = bufs_and_sems[2 * _NBUF:]

        wid = lax.axis_index("s") * _NC + lax.axis_index("c")
        b0 = wid * _BBLK

        pltpu.sync_copy(idx_hbm.at[wid], idx_v)

        def fire_gather(t, b):
            pltpu.make_async_copy(
                table_hbm.at[idx_v.at[t]], rows[b], gsem[b]).start()

        def wait_gather(b):
            pltpu.make_async_copy(
                table_hbm.at[idx_v.at[0]], rows[b], gsem[b]).wait()

        def fire_out(t, b):
            pltpu.make_async_copy(
                rows[b], out_hbm.at[pl.ds(b0, _BBLK), t], osem[b]).start()

        def wait_out(b):
            pltpu.make_async_copy(
                rows[b], out_hbm.at[pl.ds(b0, _BBLK), 0], osem[b]).wait()

        for b in range(_NBUF):
            fire_gather(b, b)
            if b >= _LAG:
                wait_gather(b - _LAG)
                fire_out(b - _LAG, b - _LAG)

        def body(g, _):
            t0 = g * _NBUF
            for b in range(_NBUF):
                t = t0 + b
                wait_out(b)
                fire_gather(t, b)
                b2 = (b - _LAG) % _NBUF
                wait_gather(b2)
                fire_out(t - _LAG, b2)
            return 0

        lax.fori_loop(1, hist // _NBUF, body, 0)

        tail0 = hist - _LAG
        for i in range(_LAG):
            b = (tail0 + i) % _NBUF
            wait_gather(b)
            fire_out(tail0 + i, b)
        for b in range(_NBUF):
            wait_out(b)

    return stage_idx, stage_gather


def kernel(x, table):
    b, h = x.shape
    d = table.shape[1]
    stage_idx, stage_gather = _build_stage(b, h, d)
    idx3 = stage_idx(jnp.transpose(x))
    return stage_gather(idx3, table)
